# baseline (speedup 1.0000x reference)
import functools

import jax
import jax.numpy as jnp
from jax import lax
from jax.experimental import pallas as pl
from jax.experimental.pallas import tpu as pltpu
from jax.experimental.pallas import tpu_sc as plsc

NC = 2
NS = 16
NW = NC * NS
L = 16

B = 16384
D = 64
CHUNK = B // NW
NIDX = 128
NJ = CHUNK // NIDX
GROUPS = CHUNK // L


def _body(uidx_hbm, midx_hbm, utab_hbm, mtab_hbm, wb_hbm, out_hbm,
          uidx_v, midx_v, urows_v, mrows_v, wb_v, out_v, ptile_v, sem):
    wid = lax.axis_index("s") * NC + lax.axis_index("c")
    base = wid * CHUNK

    pltpu.sync_copy(uidx_hbm.at[wid], uidx_v)
    pltpu.sync_copy(midx_hbm.at[wid], midx_v)
    pltpu.sync_copy(wb_hbm, wb_v)

    copies = []
    for j in range(NJ):
        copies.append(pltpu.async_copy(
            utab_hbm.at[uidx_v.at[j]], urows_v.at[pl.ds(j * NIDX, NIDX)], sem))
        copies.append(pltpu.async_copy(
            mtab_hbm.at[midx_v.at[j]], mrows_v.at[pl.ds(j * NIDX, NIDX)], sem))
    for c in copies:
        c.wait()

    wk = [wb_v[k, :] for k in range(D // L)]
    lane_iota = lax.iota(jnp.int32, L)
    bias_vec = wb_v[D // L, :]

    def group(g, carry):
        row0 = g * L
        for r in range(L):
            row = row0 + r
            acc = None
            for k in range(D // L):
                u = urows_v[row, pl.ds(L * k, L)]
                m = mrows_v[row, pl.ds(L * k, L)]
                t = (u * m) * wk[k]
                acc = t if acc is None else acc + t
            ptile_v[r, :] = acc
        s = bias_vec
        for l in range(L):
            col = plsc.load_gather(
                ptile_v, [lane_iota, jnp.full((L,), l, jnp.int32)])
            s = s + col
        out_v[pl.ds(row0, L)] = s
        return carry

    lax.fori_loop(0, GROUPS, group, 0)
    pltpu.sync_copy(out_v, out_hbm.at[pl.ds(base, CHUNK)])


@jax.jit
def _run(uidx, midx, utab, mtab, wb):
    mesh = plsc.VectorSubcoreMesh(core_axis_name="c", subcore_axis_name="s")
    kern = pl.kernel(
        _body,
        out_type=jax.ShapeDtypeStruct((B,), jnp.float32),
        mesh=mesh,
        compiler_params=pltpu.CompilerParams(
            needs_layout_passes=False, use_tc_tiling_on_sc=False),
        scratch_types=[
            pltpu.VMEM((NJ, NIDX), jnp.int32),
            pltpu.VMEM((NJ, NIDX), jnp.int32),
            pltpu.VMEM((CHUNK, D), jnp.float32),
            pltpu.VMEM((CHUNK, D), jnp.float32),
            pltpu.VMEM((D // L + 1, L), jnp.float32),
            pltpu.VMEM((CHUNK,), jnp.float32),
            pltpu.VMEM((L, L), jnp.float32),
            pltpu.SemaphoreType.DMA,
        ],
    )
    return kern(uidx, midx, utab, mtab, wb)


def kernel(user, movie, user_table, movie_table, fc_w, fc_b):
    uidx = user.astype(jnp.int32).reshape(NW, NJ, NIDX)
    midx = movie.astype(jnp.int32).reshape(NW, NJ, NIDX)
    wb = jnp.concatenate(
        [fc_w.reshape(D), jnp.broadcast_to(fc_b.reshape(1), (L,))]
    ).astype(jnp.float32).reshape(D // L + 1, L)
    out = _run(uidx, midx, user_table, movie_table, wb)
    return out.reshape(B, 1)


# speedup vs baseline: 1.6123x; 1.6123x over previous
import jax
import jax.numpy as jnp
from jax import lax
from jax.experimental import pallas as pl
from jax.experimental.pallas import tpu as pltpu
from jax.experimental.pallas import tpu_sc as plsc

NC = 2
NS = 16
NW = NC * NS
L = 16

B = 16384
D = 64
CHUNK = B // NW
NB = 128
NJ = CHUNK // NB
GPB = NB // L


def _body(uidx_hbm, midx_hbm, utab_hbm, mtab_hbm, wb_hbm, out_hbm,
          uidx_v, midx_v, urows_v, mrows_v, wb_v, out_v, ptile_v, sem):
    wid = lax.axis_index("s") * NC + lax.axis_index("c")
    base = wid * CHUNK

    pltpu.sync_copy(uidx_hbm.at[wid], uidx_v)
    pltpu.sync_copy(midx_hbm.at[wid], midx_v)
    pltpu.sync_copy(wb_hbm, wb_v)

    wk = [wb_v[k, :] for k in range(D // L)]
    lane_iota = lax.iota(jnp.int32, L)
    bias_vec = wb_v[D // L, :]

    for j in range(NJ):
        def fire(g, carry):
            row0 = g * L
            uvec = uidx_v[j, pl.ds(row0, L)]
            mvec = midx_v[j, pl.ds(row0, L)]
            for e in range(L):
                pltpu.async_copy(
                    utab_hbm.at[uvec[e]], urows_v.at[row0 + e], sem)
                pltpu.async_copy(
                    mtab_hbm.at[mvec[e]], mrows_v.at[row0 + e], sem)
            return carry

        lax.fori_loop(0, GPB, fire, 0)
        pltpu.make_async_copy(utab_hbm.at[pl.ds(0, NB)], urows_v, sem).wait()
        pltpu.make_async_copy(mtab_hbm.at[pl.ds(0, NB)], mrows_v, sem).wait()

        def group(g, carry):
            row0 = g * L
            for r in range(L):
                row = row0 + r
                acc = None
                for k in range(D // L):
                    u = urows_v[row, pl.ds(L * k, L)]
                    m = mrows_v[row, pl.ds(L * k, L)]
                    t = (u * m) * wk[k]
                    acc = t if acc is None else acc + t
                ptile_v[r, :] = acc
            s = bias_vec
            for l in range(L):
                col = plsc.load_gather(
                    ptile_v, [lane_iota, jnp.full((L,), l, jnp.int32)])
                s = s + col
            out_v[pl.ds(j * NB + row0, L)] = s
            return carry

        lax.fori_loop(0, GPB, group, 0)

    pltpu.sync_copy(out_v, out_hbm.at[pl.ds(base, CHUNK)])


@jax.jit
def _run(uidx, midx, utab, mtab, wb):
    mesh = plsc.VectorSubcoreMesh(core_axis_name="c", subcore_axis_name="s")
    kern = pl.kernel(
        _body,
        out_type=jax.ShapeDtypeStruct((B,), jnp.float32),
        mesh=mesh,
        compiler_params=pltpu.CompilerParams(
            needs_layout_passes=False, use_tc_tiling_on_sc=True),
        scratch_types=[
            pltpu.VMEM((NJ, NB), jnp.int32),
            pltpu.VMEM((NJ, NB), jnp.int32),
            pltpu.VMEM((NB, D), jnp.float32),
            pltpu.VMEM((NB, D), jnp.float32),
            pltpu.VMEM((D // L + 1, L), jnp.float32),
            pltpu.VMEM((CHUNK,), jnp.float32),
            pltpu.VMEM((L, L), jnp.float32),
            pltpu.SemaphoreType.DMA,
        ],
    )
    return kern(uidx, midx, utab, mtab, wb)


def kernel(user, movie, user_table, movie_table, fc_w, fc_b):
    uidx = user.astype(jnp.int32).reshape(NW, NJ, NB)
    midx = movie.astype(jnp.int32).reshape(NW, NJ, NB)
    wb = jnp.concatenate(
        [fc_w.reshape(D), jnp.broadcast_to(fc_b.reshape(1), (L,))]
    ).astype(jnp.float32).reshape(D // L + 1, L)
    out = _run(uidx, midx, user_table, movie_table, wb)
    return out.reshape(B, 1)
